# Initial kernel scaffold; baseline (speedup 1.0000x reference)
import jax
import jax.numpy as jnp
from jax.experimental import pallas as pl


def kernel(xyz, points, W0, gamma0, beta0, W1, gamma1, beta1, W2, gamma2, beta2):
    raise NotImplementedError("write your pallas kernel here")


# speedup vs baseline: 29.7784x; 29.7784x over previous
import functools

import jax
import jax.numpy as jnp
from jax import lax
from jax.experimental import pallas as pl
from jax.experimental.pallas import tpu as pltpu
from jax.experimental.pallas import tpu_sc as plsc

_N = 16384
_S = 512
_K = 32
import numpy as np

_R2 = np.float32(0.2 * 0.2)
_EPS = np.float32(1e-5)


def _fps_body(x_ref, y_ref, z_ref, nx_ref, ny_ref, nz_ref, dist_ref):
    B = x_ref.shape[0]
    pos = (lax.broadcasted_iota(jnp.int32, (128, 128), 0) * 128
           + lax.broadcasted_iota(jnp.int32, (128, 128), 1))
    dist_ref[...] = jnp.full((B, 128, 128), 1e10, jnp.float32)
    X = x_ref[...]
    Y = y_ref[...]
    Z = z_ref[...]

    def body(i, f):
        cmask = pos[None, :, :] == f
        cx = jnp.sum(jnp.where(cmask, X, 0.0), axis=(1, 2), keepdims=True)
        cy = jnp.sum(jnp.where(cmask, Y, 0.0), axis=(1, 2), keepdims=True)
        cz = jnp.sum(jnp.where(cmask, Z, 0.0), axis=(1, 2), keepdims=True)
        for b in range(B):
            nx_ref[b, i] = jnp.sum(cx[b])
            ny_ref[b, i] = jnp.sum(cy[b])
            nz_ref[b, i] = jnp.sum(cz[b])
        dx = X - cx
        dy = Y - cy
        dz = Z - cz
        d = (dx * dx + dy * dy) + dz * dz
        dist = jnp.minimum(dist_ref[...], d)
        dist_ref[...] = dist
        mx = jnp.max(dist, axis=(1, 2), keepdims=True)
        cand = jnp.where(dist == mx, pos[None, :, :], _N)
        return jnp.min(cand, axis=(1, 2), keepdims=True).astype(jnp.int32)

    lax.fori_loop(0, _S, body, jnp.zeros((B, 1, 1), jnp.int32))


def _run_fps(xyz):
    B = xyz.shape[0]
    x3 = xyz[..., 0].reshape(B, 128, 128)
    y3 = xyz[..., 1].reshape(B, 128, 128)
    z3 = xyz[..., 2].reshape(B, 128, 128)
    out_sds = jax.ShapeDtypeStruct((B, _S), jnp.float32)
    nx, ny, nz = pl.pallas_call(
        _fps_body,
        out_shape=(out_sds, out_sds, out_sds),
        out_specs=(pl.BlockSpec(memory_space=pltpu.SMEM),) * 3,
        scratch_shapes=[pltpu.VMEM((B, 128, 128), jnp.float32)],
    )(x3, y3, z3)
    return nx, ny, nz


def _rb(v):
    u = plsc.bitcast(v, jnp.int32)
    r = (u + ((u >> 16) & 1) + 0x7FFF) & jnp.int32(-65536)
    return plsc.bitcast(r, jnp.float32)


def _sc_body(x_hbm, y_hbm, z_hbm, nx_hbm, ny_hbm, nz_hbm, f_hbm, g_out,
             x_v, y_v, z_v, bv_v, nx_v, ny_v, nz_v, idxbuf, idx32, grow, sem):
    wid = lax.axis_index("s") * 2 + lax.axis_index("c")
    b = wid // 8
    rows = 64
    pltpu.sync_copy(x_hbm.at[pl.ds(b * _N, _N)], x_v)
    pltpu.sync_copy(y_hbm.at[pl.ds(b * _N, _N)], y_v)
    pltpu.sync_copy(z_hbm.at[pl.ds(b * _N, _N)], z_v)
    pltpu.sync_copy(nx_hbm.at[pl.ds(wid * rows * 16, rows * 16)], nx_v)
    pltpu.sync_copy(ny_hbm.at[pl.ds(wid * rows * 16, rows * 16)], ny_v)
    pltpu.sync_copy(nz_hbm.at[pl.ds(wid * rows * 16, rows * 16)], nz_v)
    iota16 = lax.iota(jnp.int32, 16)
    n16 = jnp.full((16,), _N, jnp.int32)
    two = jnp.float32(2.0)

    def pre_body(j, carry):
        px = x_v[pl.ds(j * 16, 16)]
        py = y_v[pl.ds(j * 16, 16)]
        pz = z_v[pl.ds(j * 16, 16)]
        bv_v[pl.ds(j * 16, 16)] = (px * px + py * py) + pz * pz
        x_v[pl.ds(j * 16, 16)] = _rb(px)
        y_v[pl.ds(j * 16, 16)] = _rb(py)
        z_v[pl.ds(j * 16, 16)] = _rb(pz)
        return carry

    lax.fori_loop(0, _N // 16, pre_body, jnp.int32(0))

    def row_body(r, carry):
        cxv = nx_v[pl.ds(r * 16, 16)]
        cyv = ny_v[pl.ds(r * 16, 16)]
        czv = nz_v[pl.ds(r * 16, 16)]
        a_v = (cxv * cxv + cyv * cyv) + czv * czv
        cxr = _rb(cxv)
        cyr = _rb(cyv)
        czr = _rb(czv)
        idxbuf[pl.ds(0, 16)] = n16

        def cond(state):
            j, off = state
            return jnp.logical_and(off < _K, j < _N // 16)

        def step(state):
            j, off = state
            px = x_v[pl.ds(j * 16, 16)]
            py = y_v[pl.ds(j * 16, 16)]
            pz = z_v[pl.ds(j * 16, 16)]
            b_v = bv_v[pl.ds(j * 16, 16)]
            d_v = (cxr * px + cyr * py) + czr * pz
            d2 = (a_v + b_v) - two * d_v
            m = d2 <= _R2
            plsc.store_compressed(idxbuf.at[pl.ds(off, 16)], iota16 + j * 16,
                                  mask=m)
            cnt = jnp.sum(m.astype(jnp.int32))
            return j + 1, off + cnt

        _, off = lax.while_loop(cond, step, (jnp.int32(0), jnp.int32(0)))

        v0 = idxbuf[pl.ds(0, 16)]
        v1 = idxbuf[pl.ds(16, 16)]
        fill_s = jnp.minimum(jnp.min(jnp.where(iota16 < off, v0, n16)),
                             _N - 1)
        filler = jnp.full((16,), fill_s, jnp.int32)
        boff = b * _N
        idx32[pl.ds(0, 16)] = jnp.where(iota16 < off, v0, filler) + boff
        idx32[pl.ds(16, 16)] = jnp.where(iota16 + 16 < off, v1, filler) + boff
        g_row = (wid * rows + r) * _K
        pltpu.async_copy(f_hbm.at[idx32], grow, sem).wait()
        pltpu.sync_copy(grow, g_out.at[pl.ds(g_row, _K)])
        return carry

    lax.fori_loop(0, 64, row_body, jnp.int32(0))


def _run_sc_gather(xf, yf, zf, nxf, nyf, nzf, F):
    mesh = plsc.VectorSubcoreMesh(core_axis_name="c", subcore_axis_name="s")
    rows = F.shape[0]
    call = functools.partial(
        pl.kernel,
        mesh=mesh,
        compiler_params=pltpu.CompilerParams(needs_layout_passes=False,
                                             use_tc_tiling_on_sc=False),
        out_type=jax.ShapeDtypeStruct((4 * _S * _K, 16), jnp.float32),
        scratch_types=[
            pltpu.VMEM((_N,), jnp.float32),
            pltpu.VMEM((_N,), jnp.float32),
            pltpu.VMEM((_N,), jnp.float32),
            pltpu.VMEM((_N,), jnp.float32),
            pltpu.VMEM((64 * 16,), jnp.float32),
            pltpu.VMEM((64 * 16,), jnp.float32),
            pltpu.VMEM((64 * 16,), jnp.float32),
            pltpu.VMEM((64,), jnp.int32),
            pltpu.VMEM((_K,), jnp.int32),
            pltpu.VMEM((_K, 16), jnp.float32),
            pltpu.SemaphoreType.DMA,
        ],
    )(_sc_body)
    del rows
    return call(xf, yf, zf, nxf, nyf, nzf, F)


def _bn_relu_packed(v, gamma, beta, copies, ch):
    n = jnp.float32(v.shape[0] * copies)
    s = jnp.sum(v, axis=0, keepdims=True)
    s = sum(s[:, i * ch:(i + 1) * ch] for i in range(copies))
    mean = s / n
    meanb = jnp.concatenate([mean] * copies, axis=1)
    d = v - meanb
    s2 = jnp.sum(d * d, axis=0, keepdims=True)
    s2 = sum(s2[:, i * ch:(i + 1) * ch] for i in range(copies))
    var = s2 / n
    varb = jnp.concatenate([var] * copies, axis=1)
    vh = d / jnp.sqrt(varb + _EPS)
    return jnp.maximum(vh * gamma + beta, 0.0)


def _bdot(a, w):
    return jnp.dot(a.astype(jnp.bfloat16), w.astype(jnp.bfloat16),
                   preferred_element_type=jnp.float32)


def _mlp_body(g_ref, s_ref, w0_ref, w1_ref, w2_ref,
              g1_ref, b1_ref, g2_ref, b2_ref, g3_ref, b3_ref, o_ref):
    xin = g_ref[...] - s_ref[...]
    y1 = _bdot(xin, w0_ref[...])
    h1 = _bn_relu_packed(y1, g1_ref[...], b1_ref[...], 8, 32)
    y2 = _bdot(h1, w1_ref[...])
    h2 = _bn_relu_packed(y2, g2_ref[...], b2_ref[...], 8, 32)
    y3 = _bdot(h2, w2_ref[...])
    h3 = _bn_relu_packed(y3, g3_ref[...], b3_ref[...], 8, 64)
    rows_s = h3.shape[0] // 4
    hmax = jnp.max(h3.reshape(rows_s, 4, 512), axis=1)
    m = hmax[:, 0:64]
    for i in range(1, 8):
        m = jnp.maximum(m, hmax[:, i * 64:(i + 1) * 64])
    o_ref[...] = m


def _run_mlp(Gp, Spack, W0blk, W1blk, W2blk, g1, b1, g2, b2, g3, b3):
    rows_s = Gp.shape[0] // 4
    return pl.pallas_call(
        _mlp_body,
        out_shape=jax.ShapeDtypeStruct((rows_s, 64), jnp.float32),
    )(Gp, Spack, W0blk, W1blk, W2blk, g1, b1, g2, b2, g3, b3)


@jax.jit
def kernel(xyz, points, W0, gamma0, beta0, W1, gamma1, beta1, W2, gamma2, beta2):
    B, N, _ = xyz.shape
    nx, ny, nz = _run_fps(xyz)
    new_xyz = jnp.stack([nx, ny, nz], axis=-1)

    P = jnp.concatenate([xyz, points], axis=-1).reshape(B * N, 9)
    Ppad = jnp.zeros((B * N, 16), jnp.float32).at[:, :9].set(P)

    G = _run_sc_gather(
        xyz[..., 0].reshape(-1), xyz[..., 1].reshape(-1),
        xyz[..., 2].reshape(-1),
        jnp.repeat(nx.reshape(-1), 16), jnp.repeat(ny.reshape(-1), 16),
        jnp.repeat(nz.reshape(-1), 16), Ppad)

    cpad = jnp.zeros((B * _S, 16), jnp.float32).at[:, :3].set(
        new_xyz.reshape(B * _S, 3))
    Spack = jnp.repeat(jnp.tile(cpad, (1, 8)), 4, axis=0)
    W0t = jnp.zeros((16, 32), jnp.float32).at[:9, :].set(W0.T)
    W0blk = jnp.zeros((128, 256), jnp.float32)
    W1blk = jnp.zeros((256, 256), jnp.float32)
    W2blk = jnp.zeros((256, 512), jnp.float32)
    for i in range(8):
        W0blk = W0blk.at[i * 16:(i + 1) * 16, i * 32:(i + 1) * 32].set(W0t)
        W1blk = W1blk.at[i * 32:(i + 1) * 32, i * 32:(i + 1) * 32].set(W1.T)
        W2blk = W2blk.at[i * 32:(i + 1) * 32, i * 64:(i + 1) * 64].set(W2.T)
    tile8 = lambda v: jnp.tile(v.reshape(1, -1), (1, 8))
    out = _run_mlp(G.reshape(B * _S * _K // 8, 128), Spack, W0blk,
                   W1blk, W2blk,
                   tile8(gamma0), tile8(beta0), tile8(gamma1), tile8(beta1),
                   tile8(gamma2), tile8(beta2))
    return new_xyz, out.reshape(B, _S, 64)
